# Initial kernel scaffold; baseline (speedup 1.0000x reference)
import jax
import jax.numpy as jnp
from jax.experimental import pallas as pl


def kernel(inputs, embeddings, w):
    raise NotImplementedError("write your pallas kernel here")


# speedup vs baseline: 1.0285x; 1.0285x over previous
import functools

import jax
import jax.numpy as jnp
from jax import lax
from jax.experimental import pallas as pl
from jax.experimental.pallas import tpu as pltpu
from jax.experimental.pallas import tpu_sc as plsc

VOCAB = 1000000
EMBED_DIM = 32
BATCH = 16384
N_FIELDS = 26

NW = 32
B_FLAT = BATCH * N_FIELDS
PER_W = B_FLAT // NW
SUB = 128
NSUB = 8
CHUNK = SUB * NSUB
NCHUNK = PER_W // CHUNK


def _body(idx_hbm, emb_hbm, w_hbm, out_hbm, idx_v, e_v, w_v, sem):
    wid = lax.axis_index("s") * 2 + lax.axis_index("c")

    def chunk_body(c, carry):
        pltpu.sync_copy(idx_hbm.at[wid, c], idx_v)
        cps = []
        for j in range(NSUB):
            dst = pl.ds(j * SUB, SUB)
            cps.append(pltpu.async_copy(emb_hbm.at[idx_v.at[j]], e_v.at[dst], sem))
            cps.append(pltpu.async_copy(w_hbm.at[idx_v.at[j]], w_v.at[dst], sem))
        for cp in cps:
            cp.wait()

        def mul_body(i, mcarry):
            lo = pl.ds(0, 16)
            hi = pl.ds(16, 16)
            e_v[i, lo] = e_v[i, lo] * w_v[i, lo]
            e_v[i, hi] = e_v[i, hi] * w_v[i, hi]
            return mcarry

        lax.fori_loop(0, CHUNK, mul_body, 0, unroll=4)
        pltpu.sync_copy(e_v, out_hbm.at[pl.ds(wid * PER_W + c * CHUNK, CHUNK)])
        return carry

    lax.fori_loop(0, NCHUNK, chunk_body, 0)


_lookup = functools.partial(
    pl.kernel,
    out_type=jax.ShapeDtypeStruct((B_FLAT, EMBED_DIM), jnp.float32),
    mesh=plsc.VectorSubcoreMesh(core_axis_name="c", subcore_axis_name="s"),
    scratch_types=[
        pltpu.VMEM((NSUB, SUB), jnp.int32),
        pltpu.VMEM((CHUNK, EMBED_DIM), jnp.float32),
        pltpu.VMEM((CHUNK, EMBED_DIM), jnp.float32),
        pltpu.SemaphoreType.DMA,
    ],
    compiler_params=pltpu.CompilerParams(use_tc_tiling_on_sc=False),
)(_body)


@jax.jit
def kernel(inputs, embeddings, w):
    idx = inputs.astype(jnp.int32).reshape(NW, NCHUNK, NSUB, SUB)
    out = _lookup(idx, embeddings, w)
    return out.reshape(BATCH, N_FIELDS, EMBED_DIM)
